# Initial kernel scaffold; baseline (speedup 1.0000x reference)
import jax
import jax.numpy as jnp
from jax.experimental import pallas as pl


def kernel(A, x):
    raise NotImplementedError("write your pallas kernel here")


# speedup vs baseline: 1.0062x; 1.0062x over previous
import jax
import jax.numpy as jnp
from jax.experimental import pallas as pl

M = 16384
N = 16384
BM = 256
LANES = 128


def _mv_block(a_ref, x_ref, o_ref):
    a = a_ref[...]
    x = x_ref[...]
    acc = jnp.zeros((BM, LANES), jnp.float32)
    for k in range(N // LANES):
        sl = slice(k * LANES, (k + 1) * LANES)
        acc = acc + a[:, sl] * x[:, sl]
    o_ref[...] = jnp.sum(acc, axis=1, keepdims=True)


@jax.jit
def _mv(A, x):
    out = pl.pallas_call(
        _mv_block,
        grid=(M // BM,),
        in_specs=[
            pl.BlockSpec((BM, N), lambda i: (i, 0)),
            pl.BlockSpec((1, N), lambda i: (0, 0)),
        ],
        out_specs=pl.BlockSpec((BM, 1), lambda i: (i, 0)),
        out_shape=jax.ShapeDtypeStruct((M, 1), jnp.float32),
    )(A, x.reshape(1, N))
    return out.reshape(M)


def kernel(A, x):
    return _mv(A, x)
